# baseline (speedup 1.0000x reference)
import functools

import jax
import jax.numpy as jnp
import numpy as np
from jax import lax
from jax.experimental import pallas as pl
from jax.experimental.pallas import tpu as pltpu
from jax.experimental.pallas import tpu_sc as plsc

N = 8192
C = 256
K = 16
SH = 8
CI = C // SH
PI = 6
NK = N * K
EPS = 1e-5
F32 = jnp.float32

_dot = functools.partial(jnp.dot, preferred_element_type=F32)

_PERM = np.concatenate([np.arange(0, 256, 2), np.arange(1, 256, 2)])


def _unpack(ki):
    lo = jax.lax.bitcast_convert_type(jnp.left_shift(ki, 16), F32)
    hi = jax.lax.bitcast_convert_type(
        jnp.bitwise_and(ki, jnp.int32(-65536)), F32)
    return jnp.concatenate([lo, hi], axis=1)


TD = 3 * C


def _proj(x, P):
    T = 512

    def body(x_ref, wq, wk, wv, wx, bq, bk, bv, bx, q_ref, tab_ref):
        xx = x_ref[...]
        q_ref[...] = _dot(xx, wq[...]) + bq[...]
        tab_ref[:, 0:C] = (_dot(xx, wk[...]) + bk[...]).astype(jnp.bfloat16)
        tab_ref[:, C:2 * C] = (_dot(xx, wv[...]) + bv[...]).astype(jnp.bfloat16)
        tab_ref[:, 2 * C:3 * C] = (_dot(xx, wx[...]) + bx[...]).astype(jnp.bfloat16)

    wspec = pl.BlockSpec((C, C), lambda i: (0, 0))
    bspec = pl.BlockSpec((1, C), lambda i: (0, 0))
    return pl.pallas_call(
        body,
        grid=(N // T,),
        in_specs=[pl.BlockSpec((T, C), lambda i: (i, 0))] + [wspec] * 4 + [bspec] * 4,
        out_specs=[pl.BlockSpec((T, C), lambda i: (i, 0)),
                   pl.BlockSpec((T, TD), lambda i: (i, 0))],
        out_shape=[jax.ShapeDtypeStruct((N, C), F32),
                   jax.ShapeDtypeStruct((N, TD), jnp.bfloat16)],
    )(x, P['Wq'], P['Wk'], P['Wv'], P['Wx'],
      P['bq'].reshape(1, C), P['bk'].reshape(1, C),
      P['bv'].reshape(1, C), P['bx'].reshape(1, C))


def _sc_gather(tab_i, idxf):
    TDI = TD // 2
    info = plsc.get_sparse_core_info()
    NW = info.num_cores * info.num_subcores
    BPW = NK // NW
    CH = 64
    NCH = BPW // CH
    mesh = plsc.VectorSubcoreMesh(core_axis_name="c", subcore_axis_name="s")

    @functools.partial(
        pl.kernel, mesh=mesh,
        out_type=jax.ShapeDtypeStruct((NK, TDI), jnp.int32),
        scratch_types=[
            pltpu.VMEM((BPW,), jnp.int32),
            pltpu.VMEM((CH, TDI), jnp.int32),
            pltpu.VMEM((CH, TDI), jnp.int32),
            pltpu.SemaphoreType.DMA,
            pltpu.SemaphoreType.DMA,
        ],
    )
    def k(tab_hbm, idx_hbm, out_hbm, idx_v, buf0, buf1, sem0, sem1):
        wid = lax.axis_index("s") * info.num_cores + lax.axis_index("c")
        base = wid * BPW
        pltpu.sync_copy(idx_hbm.at[pl.ds(base, BPW)], idx_v)
        pltpu.async_copy(tab_hbm.at[idx_v.at[pl.ds(0, CH)]], buf0, sem0)

        def step(i, carry):
            c0 = 2 * i

            pltpu.make_async_copy(tab_hbm.at[idx_v.at[pl.ds(0, CH)]],
                                  buf0, sem0).wait()
            pltpu.async_copy(
                tab_hbm.at[idx_v.at[pl.ds((c0 + 1) * CH, CH)]], buf1, sem1)
            pltpu.sync_copy(buf0, out_hbm.at[pl.ds(base + c0 * CH, CH)])

            pltpu.make_async_copy(tab_hbm.at[idx_v.at[pl.ds(0, CH)]],
                                  buf1, sem1).wait()

            @pl.when(i < NCH // 2 - 1)
            def _():
                pltpu.async_copy(
                    tab_hbm.at[idx_v.at[pl.ds((c0 + 2) * CH, CH)]], buf0, sem0)

            pltpu.sync_copy(buf1, out_hbm.at[pl.ds(base + (c0 + 1) * CH, CH)])
            return carry

        lax.fori_loop(0, NCH // 2, step, 0)

    return k(tab_i, idxf)


def _knn(p):
    Tq = 512
    p8 = jnp.pad(p, ((0, 0), (0, 5)))
    p8t = p8.T

    G = N // 16

    def body(pt_ref, pT_ref, idx_ref):
        pt = pt_ref[...]
        ptT = pT_ref[...]
        sq_i = jnp.sum(pt * pt, axis=1, keepdims=True)
        sq_j = jnp.sum(ptT * ptT, axis=0, keepdims=True)
        d = sq_i + sq_j - 2.0 * _dot(pt, ptT)
        chunks = [d[:, a * G:(a + 1) * G] for a in range(16)]
        big = jnp.float32(jnp.inf)
        mxi = jnp.int32(0x7FFFFFFF)
        vmins, smins = [], []
        for _ in range(4):
            r = chunks[0]
            for c in chunks[1:]:
                r = jnp.minimum(r, c)
            sl = mxi
            for a in range(15, -1, -1):
                sl = jnp.where(chunks[a] == r, jnp.int32(a), sl)
            vmins.append(r)
            smins.append(sl)
            if len(vmins) < 4:
                chunks = [jnp.where((c == r) & (sl == a), big, c)
                          for a, c in enumerate(chunks)]
        cur, n2, n3, n4 = vmins
        cs, s2, s3, s4 = smins
        biota = jax.lax.broadcasted_iota(jnp.int32, (Tq, G), 1)
        cols = []
        for t in range(K):
            m = jnp.min(cur, axis=1, keepdims=True)
            b = jnp.min(jnp.where(cur == m, biota, jnp.int32(G)),
                        axis=1, keepdims=True)
            sel = biota == b
            a_sel = jnp.min(jnp.where(sel, cs, mxi), axis=1, keepdims=True)
            cols.append(a_sel * G + b)
            if t < K - 1:
                cur = jnp.where(sel, n2, cur)
                n2 = jnp.where(sel, n3, n2)
                n3 = jnp.where(sel, n4, n3)
                n4 = jnp.where(sel, big, n4)
                cs = jnp.where(sel, s2, cs)
                s2 = jnp.where(sel, s3, s2)
                s3 = jnp.where(sel, s4, s3)
        idx_ref[...] = jnp.concatenate(cols, axis=1)

    return pl.pallas_call(
        body,
        grid=(N // Tq,),
        in_specs=[pl.BlockSpec((Tq, 8), lambda i: (i, 0)),
                  pl.BlockSpec((8, N), lambda i: (0, 0))],
        out_specs=pl.BlockSpec((Tq, K), lambda i: (i, 0)),
        out_shape=jax.ShapeDtypeStruct((N, K), jnp.int32),
    )(p8, p8t)


def _bn_coef(s, ss, cnt, g, b):
    m = s / cnt
    v = ss / cnt - m * m
    sc = g.reshape(1, -1) / jnp.sqrt(v + EPS)
    return sc, b.reshape(1, -1) - m * sc


def _p1(h_in, w, b):
    T = 8192

    def body(hin_ref, w_ref, b_ref, h_ref, s_ref, ss_ref):
        hh = _dot(hin_ref[...], w_ref[...]) + b_ref[...]
        h_ref[...] = hh

        @pl.when(pl.program_id(0) == 0)
        def _():
            s_ref[...] = jnp.zeros(s_ref.shape, F32)
            ss_ref[...] = jnp.zeros(ss_ref.shape, F32)

        s_ref[...] += jnp.sum(hh, axis=0, keepdims=True)
        ss_ref[...] += jnp.sum(hh * hh, axis=0, keepdims=True)

    stat = pl.BlockSpec((1, PI), lambda i: (0, 0))
    return pl.pallas_call(
        body,
        grid=(NK // T,),
        in_specs=[pl.BlockSpec((T, PI), lambda i: (i, 0)),
                  pl.BlockSpec((PI, PI), lambda i: (0, 0)),
                  pl.BlockSpec((1, PI), lambda i: (0, 0))],
        out_specs=[pl.BlockSpec((T, PI), lambda i: (i, 0)), stat, stat],
        out_shape=[jax.ShapeDtypeStruct((NK, PI), F32),
                   jax.ShapeDtypeStruct((1, PI), F32),
                   jax.ShapeDtypeStruct((1, PI), F32)],
    )(h_in, w, b.reshape(1, PI))


def _p2(h, kg, q, sc1, sh1, wp2, bp2):
    T = 256
    R = T * K

    def body(h_ref, kg_ref, q_ref, sc1_ref, sh1_ref, wp2_ref, bp2_ref,
             sw_ref, ssw_ref, sp_ref, ssp_ref):
        prf = _dot(jax.nn.relu(h_ref[...] * sc1_ref[...] + sh1_ref[...]),
                   wp2_ref[...]) + bp2_ref[...]
        qq = q_ref[...]
        w = (_unpack(kg_ref[...]).reshape(T, K, C) - qq[:, None, :]
             + prf.reshape(T, K, C)).reshape(R, C)

        @pl.when(pl.program_id(0) == 0)
        def _():
            sw_ref[...] = jnp.zeros(sw_ref.shape, F32)
            ssw_ref[...] = jnp.zeros(ssw_ref.shape, F32)
            sp_ref[...] = jnp.zeros(sp_ref.shape, F32)
            ssp_ref[...] = jnp.zeros(ssp_ref.shape, F32)

        sw_ref[...] += jnp.sum(w, axis=0, keepdims=True)
        ssw_ref[...] += jnp.sum(w * w, axis=0, keepdims=True)
        sp_ref[...] += jnp.sum(prf, axis=0, keepdims=True)
        ssp_ref[...] += jnp.sum(prf * prf, axis=0, keepdims=True)

    stat = pl.BlockSpec((1, C), lambda i: (0, 0))
    return pl.pallas_call(
        body,
        grid=(N // T,),
        in_specs=[pl.BlockSpec((R, PI), lambda i: (i, 0)),
                  pl.BlockSpec((R, 128), lambda i: (i, 0)),
                  pl.BlockSpec((T, C), lambda i: (i, 0)),
                  pl.BlockSpec((1, PI), lambda i: (0, 0)),
                  pl.BlockSpec((1, PI), lambda i: (0, 0)),
                  pl.BlockSpec((PI, C), lambda i: (0, 0)),
                  pl.BlockSpec((1, C), lambda i: (0, 0))],
        out_specs=[stat, stat, stat, stat],
        out_shape=[jax.ShapeDtypeStruct((1, C), F32)] * 4,
    )(h, kg, q, sc1, sh1, wp2, bp2.reshape(1, C))


def _p3(h, kg, q, sc1, sh1, wp2, bp2, scw, shw, wfw1, bbfw1, sca, sha, wl2, bl2):
    T = 256
    R = T * K

    def body(h_ref, kg_ref, q_ref, sc1_ref, sh1_ref, wp2_ref, bp2_ref,
             scw_ref, shw_ref, wfw1_ref, bbfw1_ref,
             sca_ref, sha_ref, wl2_ref, bl2_ref,
             w1_ref, h2_ref, s1_ref, ss1_ref, s2_ref, ss2_ref):
        prf = _dot(jax.nn.relu(h_ref[...] * sc1_ref[...] + sh1_ref[...]),
                   wp2_ref[...]) + bp2_ref[...]
        qq = q_ref[...]
        w = (_unpack(kg_ref[...]).reshape(T, K, C) - qq[:, None, :]
             + prf.reshape(T, K, C)).reshape(R, C)
        w1 = _dot(jax.nn.relu(w * scw_ref[...] + shw_ref[...]),
                  wfw1_ref[...]) + bbfw1_ref[...]
        h2 = _dot(jax.nn.relu(prf * sca_ref[...] + sha_ref[...]),
                  wl2_ref[...]) + bl2_ref[...]
        w1_ref[...] = w1.astype(jnp.bfloat16)
        h2_ref[...] = h2.astype(jnp.bfloat16)

        @pl.when(pl.program_id(0) == 0)
        def _():
            s1_ref[...] = jnp.zeros(s1_ref.shape, F32)
            ss1_ref[...] = jnp.zeros(ss1_ref.shape, F32)
            s2_ref[...] = jnp.zeros(s2_ref.shape, F32)
            ss2_ref[...] = jnp.zeros(ss2_ref.shape, F32)

        s1_ref[...] += jnp.sum(w1, axis=0, keepdims=True)
        ss1_ref[...] += jnp.sum(w1 * w1, axis=0, keepdims=True)
        s2_ref[...] += jnp.sum(h2, axis=0, keepdims=True)
        ss2_ref[...] += jnp.sum(h2 * h2, axis=0, keepdims=True)

    statc = pl.BlockSpec((1, C), lambda i: (0, 0))
    stati = pl.BlockSpec((1, CI), lambda i: (0, 0))
    return pl.pallas_call(
        body,
        grid=(N // T,),
        in_specs=[pl.BlockSpec((R, PI), lambda i: (i, 0)),
                  pl.BlockSpec((R, 128), lambda i: (i, 0)),
                  pl.BlockSpec((T, C), lambda i: (i, 0)),
                  pl.BlockSpec((1, PI), lambda i: (0, 0)),
                  pl.BlockSpec((1, PI), lambda i: (0, 0)),
                  pl.BlockSpec((PI, C), lambda i: (0, 0)),
                  pl.BlockSpec((1, C), lambda i: (0, 0)),
                  pl.BlockSpec((1, C), lambda i: (0, 0)),
                  pl.BlockSpec((1, C), lambda i: (0, 0)),
                  pl.BlockSpec((C, CI), lambda i: (0, 0)),
                  pl.BlockSpec((1, CI), lambda i: (0, 0)),
                  pl.BlockSpec((1, C), lambda i: (0, 0)),
                  pl.BlockSpec((1, C), lambda i: (0, 0)),
                  pl.BlockSpec((C, C), lambda i: (0, 0)),
                  pl.BlockSpec((1, C), lambda i: (0, 0))],
        out_specs=[pl.BlockSpec((R, CI), lambda i: (i, 0)),
                   pl.BlockSpec((R, C), lambda i: (i, 0)),
                   stati, stati, statc, statc],
        out_shape=[jax.ShapeDtypeStruct((NK, CI), jnp.bfloat16),
                   jax.ShapeDtypeStruct((NK, C), jnp.bfloat16),
                   jax.ShapeDtypeStruct((1, CI), F32),
                   jax.ShapeDtypeStruct((1, CI), F32),
                   jax.ShapeDtypeStruct((1, C), F32),
                   jax.ShapeDtypeStruct((1, C), F32)],
    )(h, kg, q, sc1, sh1, wp2, bp2.reshape(1, C), scw, shw,
      wfw1, bbfw1.reshape(1, CI), sca, sha, wl2, bl2.reshape(1, C))


def _p4(w1, scw2, shw2, wfw2, bbfw2, vg, h, sc1, sh1, wp2, bp2,
        h2, scb, shb, wpk, bpk, wpqm, xg, wpv, bpv, mmat):
    T = 256
    R = T * K

    def body(w1_ref, scw2_ref, shw2_ref, wfw2_ref, bbfw2_ref, vg_ref,
             h_ref, sc1_ref, sh1_ref, wp2_ref, bp2_ref,
             h2_ref, scb_ref, shb_ref, wpk_ref, bpk_ref, wpqm_ref,
             xg_ref, wpv_ref, bpv_ref, m_ref,
             feat_ref, w2_ref, vv_ref, sf_ref, ssf_ref, s2_ref, ss2_ref):
        a = _dot(jax.nn.relu(w1_ref[...].astype(F32) * scw2_ref[...]
                             + shw2_ref[...]),
                 wfw2_ref[...]) + bbfw2_ref[...]
        a3 = a.reshape(T, K, CI)
        a3 = a3 - jnp.max(a3, axis=1, keepdims=True)
        e = jnp.exp(a3)
        ws = e / jnp.sum(e, axis=1, keepdims=True)
        wst = _dot(ws.reshape(R, CI), m_ref[...]).reshape(T, K, C)
        prf = _dot(jax.nn.relu(h_ref[...] * sc1_ref[...] + sh1_ref[...]),
                   wp2_ref[...]) + bp2_ref[...]
        vp = _unpack(vg_ref[...]) + prf
        feat = jnp.sum(vp.reshape(T, K, C) * wst, axis=1)
        feat_ref[...] = feat

        p_r2 = jax.nn.relu(h2_ref[...].astype(F32) * scb_ref[...]
                           + shb_ref[...])
        kp = _dot(p_r2, wpk_ref[...]) + bpk_ref[...]
        qp = jnp.sum(p_r2 * wpqm_ref[:, 0:C], axis=1, keepdims=True) \
            + wpqm_ref[0, C]
        xg = _unpack(xg_ref[...])
        w2 = kp - qp + xg
        w2_ref[...] = w2.astype(jnp.bfloat16)
        vv_ref[...] = (_dot(p_r2, wpv_ref[...]) + bpv_ref[...]
                       + xg).astype(jnp.bfloat16)

        @pl.when(pl.program_id(0) == 0)
        def _():
            sf_ref[...] = jnp.zeros(sf_ref.shape, F32)
            ssf_ref[...] = jnp.zeros(ssf_ref.shape, F32)
            s2_ref[...] = jnp.zeros(s2_ref.shape, F32)
            ss2_ref[...] = jnp.zeros(ss2_ref.shape, F32)

        sf_ref[...] += jnp.sum(feat, axis=0, keepdims=True)
        ssf_ref[...] += jnp.sum(feat * feat, axis=0, keepdims=True)
        s2_ref[...] += jnp.sum(w2, axis=0, keepdims=True)
        ss2_ref[...] += jnp.sum(w2 * w2, axis=0, keepdims=True)

    statc = pl.BlockSpec((1, C), lambda i: (0, 0))
    cc = pl.BlockSpec((C, C), lambda i: (0, 0))
    c1 = pl.BlockSpec((1, C), lambda i: (0, 0))
    ci1 = pl.BlockSpec((1, CI), lambda i: (0, 0))
    rctile = pl.BlockSpec((R, C), lambda i: (i, 0))
    return pl.pallas_call(
        body,
        grid=(N // T,),
        in_specs=[pl.BlockSpec((R, CI), lambda i: (i, 0)), ci1, ci1,
                  pl.BlockSpec((CI, CI), lambda i: (0, 0)), ci1,
                  pl.BlockSpec((R, 128), lambda i: (i, 1)),
                  pl.BlockSpec((R, PI), lambda i: (i, 0)),
                  pl.BlockSpec((1, PI), lambda i: (0, 0)),
                  pl.BlockSpec((1, PI), lambda i: (0, 0)),
                  pl.BlockSpec((PI, C), lambda i: (0, 0)), c1,
                  pl.BlockSpec((R, C), lambda i: (i, 0)), c1, c1, cc, c1,
                  pl.BlockSpec((1, C + 128), lambda i: (0, 0)),
                  pl.BlockSpec((R, 128), lambda i: (i, 2)), cc, c1,
                  pl.BlockSpec((CI, C), lambda i: (0, 0))],
        out_specs=[pl.BlockSpec((T, C), lambda i: (i, 0)),
                   rctile, rctile, statc, statc, statc, statc],
        out_shape=[jax.ShapeDtypeStruct((N, C), F32),
                   jax.ShapeDtypeStruct((NK, C), jnp.bfloat16),
                   jax.ShapeDtypeStruct((NK, C), jnp.bfloat16),
                   jax.ShapeDtypeStruct((1, C), F32),
                   jax.ShapeDtypeStruct((1, C), F32),
                   jax.ShapeDtypeStruct((1, C), F32),
                   jax.ShapeDtypeStruct((1, C), F32)],
    )(w1, scw2, shw2, wfw2, bbfw2.reshape(1, CI), vg, h, sc1, sh1, wp2,
      bp2.reshape(1, C), h2, scb, shb, wpk, bpk.reshape(1, C), wpqm,
      xg, wpv, bpv.reshape(1, C), mmat)


def _p5(w2, sc, sh, w, bb):
    T = 256
    R = T * K

    def body(w2_ref, sc_ref, sh_ref, w_ref, bb_ref, o_ref, s_ref, ss_ref):
        w1 = _dot(jax.nn.relu(w2_ref[...].astype(F32) * sc_ref[...]
                              + sh_ref[...]),
                  w_ref[...]) + bb_ref[...]
        o_ref[...] = w1.astype(jnp.bfloat16)

        @pl.when(pl.program_id(0) == 0)
        def _():
            s_ref[...] = jnp.zeros(s_ref.shape, F32)
            ss_ref[...] = jnp.zeros(ss_ref.shape, F32)

        s_ref[...] += jnp.sum(w1, axis=0, keepdims=True)
        ss_ref[...] += jnp.sum(w1 * w1, axis=0, keepdims=True)

    stati = pl.BlockSpec((1, CI), lambda i: (0, 0))
    return pl.pallas_call(
        body,
        grid=(N // T,),
        in_specs=[pl.BlockSpec((R, C), lambda i: (i, 0)),
                  pl.BlockSpec((1, C), lambda i: (0, 0)),
                  pl.BlockSpec((1, C), lambda i: (0, 0)),
                  pl.BlockSpec((C, CI), lambda i: (0, 0)),
                  pl.BlockSpec((1, CI), lambda i: (0, 0))],
        out_specs=[pl.BlockSpec((R, CI), lambda i: (i, 0)), stati, stati],
        out_shape=[jax.ShapeDtypeStruct((NK, CI), jnp.bfloat16),
                   jax.ShapeDtypeStruct((1, CI), F32),
                   jax.ShapeDtypeStruct((1, CI), F32)],
    )(w2, sc, sh, w, bb.reshape(1, CI))


def _p6(w21, sc, sh, w, bb, vv, mmat):
    T = 256
    R = T * K

    def body(w21_ref, sc_ref, sh_ref, w_ref, bb_ref, vv_ref, m_ref,
             post_ref, s_ref, ss_ref):
        a = _dot(jax.nn.relu(w21_ref[...].astype(F32) * sc_ref[...]
                             + sh_ref[...]),
                 w_ref[...]) + bb_ref[...]
        a3 = a.reshape(T, K, CI)
        a3 = a3 - jnp.max(a3, axis=1, keepdims=True)
        e = jnp.exp(a3)
        ws = e / jnp.sum(e, axis=1, keepdims=True)
        wst = _dot(ws.reshape(R, CI), m_ref[...]).reshape(T, K, C)
        post = jnp.sum(vv_ref[...].astype(F32).reshape(T, K, C) * wst, axis=1)
        post_ref[...] = post

        @pl.when(pl.program_id(0) == 0)
        def _():
            s_ref[...] = jnp.zeros(s_ref.shape, F32)
            ss_ref[...] = jnp.zeros(ss_ref.shape, F32)

        s_ref[...] += jnp.sum(post, axis=0, keepdims=True)
        ss_ref[...] += jnp.sum(post * post, axis=0, keepdims=True)

    statc = pl.BlockSpec((1, C), lambda i: (0, 0))
    return pl.pallas_call(
        body,
        grid=(N // T,),
        in_specs=[pl.BlockSpec((R, CI), lambda i: (i, 0)),
                  pl.BlockSpec((1, CI), lambda i: (0, 0)),
                  pl.BlockSpec((1, CI), lambda i: (0, 0)),
                  pl.BlockSpec((CI, CI), lambda i: (0, 0)),
                  pl.BlockSpec((1, CI), lambda i: (0, 0)),
                  pl.BlockSpec((R, C), lambda i: (i, 0)),
                  pl.BlockSpec((CI, C), lambda i: (0, 0))],
        out_specs=[pl.BlockSpec((T, C), lambda i: (i, 0)), statc, statc],
        out_shape=[jax.ShapeDtypeStruct((N, C), F32),
                   jax.ShapeDtypeStruct((1, C), F32),
                   jax.ShapeDtypeStruct((1, C), F32)],
    )(w21, sc, sh, w, bb.reshape(1, CI), vv, mmat)


def _p7(feat, scf, shf, post, scp, shp, wfpa, wfpb, bfp):
    T = 512

    def body(f_ref, scf_ref, shf_ref, p_ref, scp_ref, shp_ref,
             wa_ref, wb_ref, b_ref, o_ref):
        f = jax.nn.relu(f_ref[...] * scf_ref[...] + shf_ref[...])
        pp = jax.nn.relu(p_ref[...] * scp_ref[...] + shp_ref[...])
        o_ref[...] = _dot(f, wa_ref[...]) + _dot(pp, wb_ref[...]) + b_ref[...]

    c1 = pl.BlockSpec((1, C), lambda i: (0, 0))
    cc = pl.BlockSpec((C, C), lambda i: (0, 0))
    return pl.pallas_call(
        body,
        grid=(N // T,),
        in_specs=[pl.BlockSpec((T, C), lambda i: (i, 0)), c1, c1,
                  pl.BlockSpec((T, C), lambda i: (i, 0)), c1, c1,
                  cc, cc, c1],
        out_specs=pl.BlockSpec((T, C), lambda i: (i, 0)),
        out_shape=jax.ShapeDtypeStruct((N, C), F32),
    )(feat, scf, shf, post, scp, shp, wfpa, wfpb, bfp.reshape(1, C))


def _sphere(p_r):
    rho = jnp.sqrt(jnp.sum(p_r * p_r, axis=-1, keepdims=True))
    rho_s = jnp.where(rho == 0, 1.0, rho)
    theta = jnp.arccos(jnp.clip(p_r[..., 2:3] / rho_s, -1.0, 1.0))
    theta = jnp.where(rho == 0, 0.0, theta)
    phi = jnp.arctan2(p_r[..., 1:2], p_r[..., 0:1])
    return jnp.concatenate([rho, theta / jnp.pi, phi / (2 * jnp.pi) + 0.5],
                           axis=-1)


def kernel(p, x, o, params):
    P = params
    pm = _PERM
    mmat = jnp.asarray(
        (np.arange(CI)[:, None] == (pm % CI)[None, :]).astype(np.float32))

    Pq = dict(P)
    Pq['Wq'] = P['Wq'][:, pm]
    Pq['bq'] = P['bq'][pm]
    q, tab = _proj(x, Pq)
    idx = _knn(p)
    idxf = idx.reshape(-1)

    tab_i = lax.bitcast_convert_type(tab.reshape(N, TD // 2, 2), jnp.int32)
    g = _sc_gather(tab_i, idxf)
    p_r = jnp.take(p, idxf, axis=0) \
        - jnp.broadcast_to(p[:, None, :], (N, K, 3)).reshape(NK, 3)
    h_in = jnp.concatenate([p_r, _sphere(p_r)], axis=-1)

    wp2p = P['Wp2'][:, pm]
    bp2p = P['bp2'][pm]

    h, s_h, ss_h = _p1(h_in, P['Wp1'], P['bp1'])
    sc1, sh1 = _bn_coef(s_h, ss_h, NK, P['g_p1'], P['b_p1'])

    sw, ssw, sp, ssp = _p2(h, g, q, sc1, sh1, wp2p, bp2p)
    scw, shw = _bn_coef(sw, ssw, NK, P['g_fw1'][pm], P['b_fw1'][pm])
    sca, sha = _bn_coef(sp, ssp, NK, P['g_l2a'][pm], P['b_l2a'][pm])

    w1, h2, s1, ss1, s2, ss2 = _p3(h, g, q, sc1, sh1, wp2p, bp2p,
                                   scw, shw, P['Wfw1'][pm], P['bb_fw1'],
                                   sca, sha, P['Wl2'][pm], P['bl2'])
    scw2, shw2 = _bn_coef(s1, ss1, NK, P['g_fw2'], P['b_fw2'])
    scb, shb = _bn_coef(s2, ss2, NK, P['g_l2b'], P['b_l2b'])

    wpqm = jnp.concatenate([jnp.mean(P['Wpq'], axis=1).reshape(1, C),
                            jnp.full((1, 128), jnp.mean(P['bpq']), F32)],
                           axis=1)
    feat, w2, vv, sf, ssf, sw2, ssw2 = _p4(
        w1, scw2, shw2, P['Wfw2'], P['bb_fw2'], g, h, sc1, sh1,
        wp2p, bp2p, h2, scb, shb, P['Wpk'][:, pm], P['bpk'][pm], wpqm,
        g, P['Wpv'][:, pm], P['bpv'][pm], mmat)
    scf, shf = _bn_coef(sf, ssf, N, P['g_brf'][pm], P['b_brf'][pm])
    scpw, shpw = _bn_coef(sw2, ssw2, NK, P['g_pw1'][pm], P['b_pw1'][pm])

    w21, s21, ss21 = _p5(w2, scpw, shpw, P['Wpw1'][pm], P['bb_pw1'])
    scpw2, shpw2 = _bn_coef(s21, ss21, NK, P['g_pw2'], P['b_pw2'])

    post, spo, sspo = _p6(w21, scpw2, shpw2, P['Wpw2'], P['bb_pw2'], vv, mmat)
    scpo, shpo = _bn_coef(spo, sspo, N, P['g_brp'][pm], P['b_brp'][pm])

    return _p7(feat, scf, shf, post, scpo, shpo,
               P['Wfp'][0:C][pm], P['Wfp'][C:2 * C][pm], P['bfp'])


# speedup vs baseline: 1.0103x; 1.0103x over previous
import functools

import jax
import jax.numpy as jnp
import numpy as np
from jax import lax
from jax.experimental import pallas as pl
from jax.experimental.pallas import tpu as pltpu
from jax.experimental.pallas import tpu_sc as plsc

N = 8192
C = 256
K = 16
SH = 8
CI = C // SH
PI = 6
NK = N * K
EPS = 1e-5
F32 = jnp.float32

_dot = functools.partial(jnp.dot, preferred_element_type=F32)

_PERM = np.concatenate([np.arange(0, 256, 2), np.arange(1, 256, 2)])


def _unpack(ki):
    lo = jax.lax.bitcast_convert_type(jnp.left_shift(ki, 16), F32)
    hi = jax.lax.bitcast_convert_type(
        jnp.bitwise_and(ki, jnp.int32(-65536)), F32)
    return jnp.concatenate([lo, hi], axis=1)


TD = 3 * C


def _proj(x, P):
    T = 512

    def body(x_ref, wq, wk, wv, wx, bq, bk, bv, bx, q_ref, tab_ref):
        xx = x_ref[...]
        q_ref[...] = _dot(xx, wq[...]) + bq[...]
        tab_ref[:, 0:C] = (_dot(xx, wk[...]) + bk[...]).astype(jnp.bfloat16)
        tab_ref[:, C:2 * C] = (_dot(xx, wv[...]) + bv[...]).astype(jnp.bfloat16)
        tab_ref[:, 2 * C:3 * C] = (_dot(xx, wx[...]) + bx[...]).astype(jnp.bfloat16)

    wspec = pl.BlockSpec((C, C), lambda i: (0, 0))
    bspec = pl.BlockSpec((1, C), lambda i: (0, 0))
    return pl.pallas_call(
        body,
        grid=(N // T,),
        in_specs=[pl.BlockSpec((T, C), lambda i: (i, 0))] + [wspec] * 4 + [bspec] * 4,
        out_specs=[pl.BlockSpec((T, C), lambda i: (i, 0)),
                   pl.BlockSpec((T, TD), lambda i: (i, 0))],
        out_shape=[jax.ShapeDtypeStruct((N, C), F32),
                   jax.ShapeDtypeStruct((N, TD), jnp.bfloat16)],
    )(x, P['Wq'], P['Wk'], P['Wv'], P['Wx'],
      P['bq'].reshape(1, C), P['bk'].reshape(1, C),
      P['bv'].reshape(1, C), P['bx'].reshape(1, C))


def _sc_gather(tab_i, idxf):
    TDI = TD // 2
    info = plsc.get_sparse_core_info()
    NW = info.num_cores * info.num_subcores
    BPW = NK // NW
    CH = 64
    NCH = BPW // CH
    mesh = plsc.VectorSubcoreMesh(core_axis_name="c", subcore_axis_name="s")

    @functools.partial(
        pl.kernel, mesh=mesh,
        out_type=jax.ShapeDtypeStruct((NK, TDI), jnp.int32),
        scratch_types=[
            pltpu.VMEM((BPW,), jnp.int32),
            pltpu.VMEM((CH, TDI), jnp.int32),
            pltpu.VMEM((CH, TDI), jnp.int32),
            pltpu.SemaphoreType.DMA,
            pltpu.SemaphoreType.DMA,
        ],
    )
    def k(tab_hbm, idx_hbm, out_hbm, idx_v, buf0, buf1, sem0, sem1):
        wid = lax.axis_index("s") * info.num_cores + lax.axis_index("c")
        base = wid * BPW
        pltpu.sync_copy(idx_hbm.at[pl.ds(base, BPW)], idx_v)
        pltpu.async_copy(tab_hbm.at[idx_v.at[pl.ds(0, CH)]], buf0, sem0)

        def step(i, carry):
            c0 = 2 * i

            pltpu.make_async_copy(tab_hbm.at[idx_v.at[pl.ds(0, CH)]],
                                  buf0, sem0).wait()
            pltpu.async_copy(
                tab_hbm.at[idx_v.at[pl.ds((c0 + 1) * CH, CH)]], buf1, sem1)
            pltpu.sync_copy(buf0, out_hbm.at[pl.ds(base + c0 * CH, CH)])

            pltpu.make_async_copy(tab_hbm.at[idx_v.at[pl.ds(0, CH)]],
                                  buf1, sem1).wait()

            @pl.when(i < NCH // 2 - 1)
            def _():
                pltpu.async_copy(
                    tab_hbm.at[idx_v.at[pl.ds((c0 + 2) * CH, CH)]], buf0, sem0)

            pltpu.sync_copy(buf1, out_hbm.at[pl.ds(base + (c0 + 1) * CH, CH)])
            return carry

        lax.fori_loop(0, NCH // 2, step, 0)

    return k(tab_i, idxf)


def _knn(p):
    Tq = 512
    p8 = jnp.pad(p, ((0, 0), (0, 5)))
    p8t = p8.T

    G = N // 16

    def body(pt_ref, pT_ref, idx_ref):
        pt = pt_ref[...]
        ptT = pT_ref[...]
        sq_i = jnp.sum(pt * pt, axis=1, keepdims=True)
        sq_j = jnp.sum(ptT * ptT, axis=0, keepdims=True)
        d = sq_i + sq_j - 2.0 * _dot(pt, ptT)
        chunks = [d[:, a * G:(a + 1) * G] for a in range(16)]
        big = jnp.float32(jnp.inf)
        mxi = jnp.int32(0x7FFFFFFF)
        vmins, smins = [], []
        for _ in range(4):
            r = chunks[0]
            for c in chunks[1:]:
                r = jnp.minimum(r, c)
            sl = mxi
            for a in range(15, -1, -1):
                sl = jnp.where(chunks[a] == r, jnp.int32(a), sl)
            vmins.append(r)
            smins.append(sl)
            if len(vmins) < 4:
                chunks = [jnp.where((c == r) & (sl == a), big, c)
                          for a, c in enumerate(chunks)]
        cur, n2, n3, n4 = vmins
        cs, s2, s3, s4 = smins
        biota = jax.lax.broadcasted_iota(jnp.int32, (Tq, G), 1)
        cols = []
        for t in range(K):
            m = jnp.min(cur, axis=1, keepdims=True)
            j = jnp.min(jnp.where(cur == m, cs * G + biota, mxi),
                        axis=1, keepdims=True)
            cols.append(j)
            sel = biota == jnp.bitwise_and(j, jnp.int32(G - 1))
            if t < K - 1:
                cur = jnp.where(sel, n2, cur)
                n2 = jnp.where(sel, n3, n2)
                n3 = jnp.where(sel, n4, n3)
                n4 = jnp.where(sel, big, n4)
                cs = jnp.where(sel, s2, cs)
                s2 = jnp.where(sel, s3, s2)
                s3 = jnp.where(sel, s4, s3)
        idx_ref[...] = jnp.concatenate(cols, axis=1)

    return pl.pallas_call(
        body,
        grid=(N // Tq,),
        in_specs=[pl.BlockSpec((Tq, 8), lambda i: (i, 0)),
                  pl.BlockSpec((8, N), lambda i: (0, 0))],
        out_specs=pl.BlockSpec((Tq, K), lambda i: (i, 0)),
        out_shape=jax.ShapeDtypeStruct((N, K), jnp.int32),
    )(p8, p8t)


def _bn_coef(s, ss, cnt, g, b):
    m = s / cnt
    v = ss / cnt - m * m
    sc = g.reshape(1, -1) / jnp.sqrt(v + EPS)
    return sc, b.reshape(1, -1) - m * sc


def _p1(h_in, w, b):
    T = 8192

    def body(hin_ref, w_ref, b_ref, h_ref, s_ref, ss_ref):
        hh = _dot(hin_ref[...], w_ref[...]) + b_ref[...]
        h_ref[...] = hh

        @pl.when(pl.program_id(0) == 0)
        def _():
            s_ref[...] = jnp.zeros(s_ref.shape, F32)
            ss_ref[...] = jnp.zeros(ss_ref.shape, F32)

        s_ref[...] += jnp.sum(hh, axis=0, keepdims=True)
        ss_ref[...] += jnp.sum(hh * hh, axis=0, keepdims=True)

    stat = pl.BlockSpec((1, PI), lambda i: (0, 0))
    return pl.pallas_call(
        body,
        grid=(NK // T,),
        in_specs=[pl.BlockSpec((T, PI), lambda i: (i, 0)),
                  pl.BlockSpec((PI, PI), lambda i: (0, 0)),
                  pl.BlockSpec((1, PI), lambda i: (0, 0))],
        out_specs=[pl.BlockSpec((T, PI), lambda i: (i, 0)), stat, stat],
        out_shape=[jax.ShapeDtypeStruct((NK, PI), F32),
                   jax.ShapeDtypeStruct((1, PI), F32),
                   jax.ShapeDtypeStruct((1, PI), F32)],
    )(h_in, w, b.reshape(1, PI))


def _p2(h, kg, q, sc1, sh1, wp2, bp2):
    T = 256
    R = T * K

    def body(h_ref, kg_ref, q_ref, sc1_ref, sh1_ref, wp2_ref, bp2_ref,
             sw_ref, ssw_ref, sp_ref, ssp_ref):
        prf = _dot(jax.nn.relu(h_ref[...] * sc1_ref[...] + sh1_ref[...]),
                   wp2_ref[...]) + bp2_ref[...]
        qq = q_ref[...]
        w = (_unpack(kg_ref[...]).reshape(T, K, C) - qq[:, None, :]
             + prf.reshape(T, K, C)).reshape(R, C)

        @pl.when(pl.program_id(0) == 0)
        def _():
            sw_ref[...] = jnp.zeros(sw_ref.shape, F32)
            ssw_ref[...] = jnp.zeros(ssw_ref.shape, F32)
            sp_ref[...] = jnp.zeros(sp_ref.shape, F32)
            ssp_ref[...] = jnp.zeros(ssp_ref.shape, F32)

        sw_ref[...] += jnp.sum(w, axis=0, keepdims=True)
        ssw_ref[...] += jnp.sum(w * w, axis=0, keepdims=True)
        sp_ref[...] += jnp.sum(prf, axis=0, keepdims=True)
        ssp_ref[...] += jnp.sum(prf * prf, axis=0, keepdims=True)

    stat = pl.BlockSpec((1, C), lambda i: (0, 0))
    return pl.pallas_call(
        body,
        grid=(N // T,),
        in_specs=[pl.BlockSpec((R, PI), lambda i: (i, 0)),
                  pl.BlockSpec((R, 128), lambda i: (i, 0)),
                  pl.BlockSpec((T, C), lambda i: (i, 0)),
                  pl.BlockSpec((1, PI), lambda i: (0, 0)),
                  pl.BlockSpec((1, PI), lambda i: (0, 0)),
                  pl.BlockSpec((PI, C), lambda i: (0, 0)),
                  pl.BlockSpec((1, C), lambda i: (0, 0))],
        out_specs=[stat, stat, stat, stat],
        out_shape=[jax.ShapeDtypeStruct((1, C), F32)] * 4,
    )(h, kg, q, sc1, sh1, wp2, bp2.reshape(1, C))


def _p3(h, kg, q, sc1, sh1, wp2, bp2, scw, shw, wfw1, bbfw1, sca, sha, wl2, bl2):
    T = 256
    R = T * K

    def body(h_ref, kg_ref, q_ref, sc1_ref, sh1_ref, wp2_ref, bp2_ref,
             scw_ref, shw_ref, wfw1_ref, bbfw1_ref,
             sca_ref, sha_ref, wl2_ref, bl2_ref,
             w1_ref, h2_ref, s1_ref, ss1_ref, s2_ref, ss2_ref):
        prf = _dot(jax.nn.relu(h_ref[...] * sc1_ref[...] + sh1_ref[...]),
                   wp2_ref[...]) + bp2_ref[...]
        qq = q_ref[...]
        w = (_unpack(kg_ref[...]).reshape(T, K, C) - qq[:, None, :]
             + prf.reshape(T, K, C)).reshape(R, C)
        w1 = _dot(jax.nn.relu(w * scw_ref[...] + shw_ref[...]),
                  wfw1_ref[...]) + bbfw1_ref[...]
        h2 = _dot(jax.nn.relu(prf * sca_ref[...] + sha_ref[...]),
                  wl2_ref[...]) + bl2_ref[...]
        w1_ref[...] = w1.astype(jnp.bfloat16)
        h2_ref[...] = h2.astype(jnp.bfloat16)

        @pl.when(pl.program_id(0) == 0)
        def _():
            s1_ref[...] = jnp.zeros(s1_ref.shape, F32)
            ss1_ref[...] = jnp.zeros(ss1_ref.shape, F32)
            s2_ref[...] = jnp.zeros(s2_ref.shape, F32)
            ss2_ref[...] = jnp.zeros(ss2_ref.shape, F32)

        s1_ref[...] += jnp.sum(w1, axis=0, keepdims=True)
        ss1_ref[...] += jnp.sum(w1 * w1, axis=0, keepdims=True)
        s2_ref[...] += jnp.sum(h2, axis=0, keepdims=True)
        ss2_ref[...] += jnp.sum(h2 * h2, axis=0, keepdims=True)

    statc = pl.BlockSpec((1, C), lambda i: (0, 0))
    stati = pl.BlockSpec((1, CI), lambda i: (0, 0))
    return pl.pallas_call(
        body,
        grid=(N // T,),
        in_specs=[pl.BlockSpec((R, PI), lambda i: (i, 0)),
                  pl.BlockSpec((R, 128), lambda i: (i, 0)),
                  pl.BlockSpec((T, C), lambda i: (i, 0)),
                  pl.BlockSpec((1, PI), lambda i: (0, 0)),
                  pl.BlockSpec((1, PI), lambda i: (0, 0)),
                  pl.BlockSpec((PI, C), lambda i: (0, 0)),
                  pl.BlockSpec((1, C), lambda i: (0, 0)),
                  pl.BlockSpec((1, C), lambda i: (0, 0)),
                  pl.BlockSpec((1, C), lambda i: (0, 0)),
                  pl.BlockSpec((C, CI), lambda i: (0, 0)),
                  pl.BlockSpec((1, CI), lambda i: (0, 0)),
                  pl.BlockSpec((1, C), lambda i: (0, 0)),
                  pl.BlockSpec((1, C), lambda i: (0, 0)),
                  pl.BlockSpec((C, C), lambda i: (0, 0)),
                  pl.BlockSpec((1, C), lambda i: (0, 0))],
        out_specs=[pl.BlockSpec((R, CI), lambda i: (i, 0)),
                   pl.BlockSpec((R, C), lambda i: (i, 0)),
                   stati, stati, statc, statc],
        out_shape=[jax.ShapeDtypeStruct((NK, CI), jnp.bfloat16),
                   jax.ShapeDtypeStruct((NK, C), jnp.bfloat16),
                   jax.ShapeDtypeStruct((1, CI), F32),
                   jax.ShapeDtypeStruct((1, CI), F32),
                   jax.ShapeDtypeStruct((1, C), F32),
                   jax.ShapeDtypeStruct((1, C), F32)],
    )(h, kg, q, sc1, sh1, wp2, bp2.reshape(1, C), scw, shw,
      wfw1, bbfw1.reshape(1, CI), sca, sha, wl2, bl2.reshape(1, C))


def _p4(w1, scw2, shw2, wfw2, bbfw2, vg, h, sc1, sh1, wp2, bp2,
        h2, scb, shb, wpk, bpk, wpqm, xg, wpv, bpv, mmat):
    T = 256
    R = T * K

    def body(w1_ref, scw2_ref, shw2_ref, wfw2_ref, bbfw2_ref, vg_ref,
             h_ref, sc1_ref, sh1_ref, wp2_ref, bp2_ref,
             h2_ref, scb_ref, shb_ref, wpk_ref, bpk_ref, wpqm_ref,
             xg_ref, wpv_ref, bpv_ref, m_ref,
             feat_ref, w2_ref, vv_ref, sf_ref, ssf_ref, s2_ref, ss2_ref):
        a = _dot(jax.nn.relu(w1_ref[...].astype(F32) * scw2_ref[...]
                             + shw2_ref[...]),
                 wfw2_ref[...]) + bbfw2_ref[...]
        a3 = a.reshape(T, K, CI)
        a3 = a3 - jnp.max(a3, axis=1, keepdims=True)
        e = jnp.exp(a3)
        ws = e / jnp.sum(e, axis=1, keepdims=True)
        wst = _dot(ws.reshape(R, CI), m_ref[...]).reshape(T, K, C)
        prf = _dot(jax.nn.relu(h_ref[...] * sc1_ref[...] + sh1_ref[...]),
                   wp2_ref[...]) + bp2_ref[...]
        vp = _unpack(vg_ref[...]) + prf
        feat = jnp.sum(vp.reshape(T, K, C) * wst, axis=1)
        feat_ref[...] = feat

        p_r2 = jax.nn.relu(h2_ref[...].astype(F32) * scb_ref[...]
                           + shb_ref[...])
        kp = _dot(p_r2, wpk_ref[...]) + bpk_ref[...]
        qp = jnp.sum(p_r2 * wpqm_ref[:, 0:C], axis=1, keepdims=True) \
            + wpqm_ref[0, C]
        xg = _unpack(xg_ref[...])
        w2 = kp - qp + xg
        w2_ref[...] = w2.astype(jnp.bfloat16)
        vv_ref[...] = (_dot(p_r2, wpv_ref[...]) + bpv_ref[...]
                       + xg).astype(jnp.bfloat16)

        @pl.when(pl.program_id(0) == 0)
        def _():
            sf_ref[...] = jnp.zeros(sf_ref.shape, F32)
            ssf_ref[...] = jnp.zeros(ssf_ref.shape, F32)
            s2_ref[...] = jnp.zeros(s2_ref.shape, F32)
            ss2_ref[...] = jnp.zeros(ss2_ref.shape, F32)

        sf_ref[...] += jnp.sum(feat, axis=0, keepdims=True)
        ssf_ref[...] += jnp.sum(feat * feat, axis=0, keepdims=True)
        s2_ref[...] += jnp.sum(w2, axis=0, keepdims=True)
        ss2_ref[...] += jnp.sum(w2 * w2, axis=0, keepdims=True)

    statc = pl.BlockSpec((1, C), lambda i: (0, 0))
    cc = pl.BlockSpec((C, C), lambda i: (0, 0))
    c1 = pl.BlockSpec((1, C), lambda i: (0, 0))
    ci1 = pl.BlockSpec((1, CI), lambda i: (0, 0))
    rctile = pl.BlockSpec((R, C), lambda i: (i, 0))
    return pl.pallas_call(
        body,
        grid=(N // T,),
        in_specs=[pl.BlockSpec((R, CI), lambda i: (i, 0)), ci1, ci1,
                  pl.BlockSpec((CI, CI), lambda i: (0, 0)), ci1,
                  pl.BlockSpec((R, 128), lambda i: (i, 1)),
                  pl.BlockSpec((R, PI), lambda i: (i, 0)),
                  pl.BlockSpec((1, PI), lambda i: (0, 0)),
                  pl.BlockSpec((1, PI), lambda i: (0, 0)),
                  pl.BlockSpec((PI, C), lambda i: (0, 0)), c1,
                  pl.BlockSpec((R, C), lambda i: (i, 0)), c1, c1, cc, c1,
                  pl.BlockSpec((1, C + 128), lambda i: (0, 0)),
                  pl.BlockSpec((R, 128), lambda i: (i, 2)), cc, c1,
                  pl.BlockSpec((CI, C), lambda i: (0, 0))],
        out_specs=[pl.BlockSpec((T, C), lambda i: (i, 0)),
                   rctile, rctile, statc, statc, statc, statc],
        out_shape=[jax.ShapeDtypeStruct((N, C), F32),
                   jax.ShapeDtypeStruct((NK, C), jnp.bfloat16),
                   jax.ShapeDtypeStruct((NK, C), jnp.bfloat16),
                   jax.ShapeDtypeStruct((1, C), F32),
                   jax.ShapeDtypeStruct((1, C), F32),
                   jax.ShapeDtypeStruct((1, C), F32),
                   jax.ShapeDtypeStruct((1, C), F32)],
    )(w1, scw2, shw2, wfw2, bbfw2.reshape(1, CI), vg, h, sc1, sh1, wp2,
      bp2.reshape(1, C), h2, scb, shb, wpk, bpk.reshape(1, C), wpqm,
      xg, wpv, bpv.reshape(1, C), mmat)


def _p5(w2, sc, sh, w, bb):
    T = 256
    R = T * K

    def body(w2_ref, sc_ref, sh_ref, w_ref, bb_ref, o_ref, s_ref, ss_ref):
        w1 = _dot(jax.nn.relu(w2_ref[...].astype(F32) * sc_ref[...]
                              + sh_ref[...]),
                  w_ref[...]) + bb_ref[...]
        o_ref[...] = w1.astype(jnp.bfloat16)

        @pl.when(pl.program_id(0) == 0)
        def _():
            s_ref[...] = jnp.zeros(s_ref.shape, F32)
            ss_ref[...] = jnp.zeros(ss_ref.shape, F32)

        s_ref[...] += jnp.sum(w1, axis=0, keepdims=True)
        ss_ref[...] += jnp.sum(w1 * w1, axis=0, keepdims=True)

    stati = pl.BlockSpec((1, CI), lambda i: (0, 0))
    return pl.pallas_call(
        body,
        grid=(N // T,),
        in_specs=[pl.BlockSpec((R, C), lambda i: (i, 0)),
                  pl.BlockSpec((1, C), lambda i: (0, 0)),
                  pl.BlockSpec((1, C), lambda i: (0, 0)),
                  pl.BlockSpec((C, CI), lambda i: (0, 0)),
                  pl.BlockSpec((1, CI), lambda i: (0, 0))],
        out_specs=[pl.BlockSpec((R, CI), lambda i: (i, 0)), stati, stati],
        out_shape=[jax.ShapeDtypeStruct((NK, CI), jnp.bfloat16),
                   jax.ShapeDtypeStruct((1, CI), F32),
                   jax.ShapeDtypeStruct((1, CI), F32)],
    )(w2, sc, sh, w, bb.reshape(1, CI))


def _p6(w21, sc, sh, w, bb, vv, mmat):
    T = 256
    R = T * K

    def body(w21_ref, sc_ref, sh_ref, w_ref, bb_ref, vv_ref, m_ref,
             post_ref, s_ref, ss_ref):
        a = _dot(jax.nn.relu(w21_ref[...].astype(F32) * sc_ref[...]
                             + sh_ref[...]),
                 w_ref[...]) + bb_ref[...]
        a3 = a.reshape(T, K, CI)
        a3 = a3 - jnp.max(a3, axis=1, keepdims=True)
        e = jnp.exp(a3)
        ws = e / jnp.sum(e, axis=1, keepdims=True)
        wst = _dot(ws.reshape(R, CI), m_ref[...]).reshape(T, K, C)
        post = jnp.sum(vv_ref[...].astype(F32).reshape(T, K, C) * wst, axis=1)
        post_ref[...] = post

        @pl.when(pl.program_id(0) == 0)
        def _():
            s_ref[...] = jnp.zeros(s_ref.shape, F32)
            ss_ref[...] = jnp.zeros(ss_ref.shape, F32)

        s_ref[...] += jnp.sum(post, axis=0, keepdims=True)
        ss_ref[...] += jnp.sum(post * post, axis=0, keepdims=True)

    statc = pl.BlockSpec((1, C), lambda i: (0, 0))
    return pl.pallas_call(
        body,
        grid=(N // T,),
        in_specs=[pl.BlockSpec((R, CI), lambda i: (i, 0)),
                  pl.BlockSpec((1, CI), lambda i: (0, 0)),
                  pl.BlockSpec((1, CI), lambda i: (0, 0)),
                  pl.BlockSpec((CI, CI), lambda i: (0, 0)),
                  pl.BlockSpec((1, CI), lambda i: (0, 0)),
                  pl.BlockSpec((R, C), lambda i: (i, 0)),
                  pl.BlockSpec((CI, C), lambda i: (0, 0))],
        out_specs=[pl.BlockSpec((T, C), lambda i: (i, 0)), statc, statc],
        out_shape=[jax.ShapeDtypeStruct((N, C), F32),
                   jax.ShapeDtypeStruct((1, C), F32),
                   jax.ShapeDtypeStruct((1, C), F32)],
    )(w21, sc, sh, w, bb.reshape(1, CI), vv, mmat)


def _p7(feat, scf, shf, post, scp, shp, wfpa, wfpb, bfp):
    T = 512

    def body(f_ref, scf_ref, shf_ref, p_ref, scp_ref, shp_ref,
             wa_ref, wb_ref, b_ref, o_ref):
        f = jax.nn.relu(f_ref[...] * scf_ref[...] + shf_ref[...])
        pp = jax.nn.relu(p_ref[...] * scp_ref[...] + shp_ref[...])
        o_ref[...] = _dot(f, wa_ref[...]) + _dot(pp, wb_ref[...]) + b_ref[...]

    c1 = pl.BlockSpec((1, C), lambda i: (0, 0))
    cc = pl.BlockSpec((C, C), lambda i: (0, 0))
    return pl.pallas_call(
        body,
        grid=(N // T,),
        in_specs=[pl.BlockSpec((T, C), lambda i: (i, 0)), c1, c1,
                  pl.BlockSpec((T, C), lambda i: (i, 0)), c1, c1,
                  cc, cc, c1],
        out_specs=pl.BlockSpec((T, C), lambda i: (i, 0)),
        out_shape=jax.ShapeDtypeStruct((N, C), F32),
    )(feat, scf, shf, post, scp, shp, wfpa, wfpb, bfp.reshape(1, C))


def _sphere(p_r):
    rho = jnp.sqrt(jnp.sum(p_r * p_r, axis=-1, keepdims=True))
    rho_s = jnp.where(rho == 0, 1.0, rho)
    theta = jnp.arccos(jnp.clip(p_r[..., 2:3] / rho_s, -1.0, 1.0))
    theta = jnp.where(rho == 0, 0.0, theta)
    phi = jnp.arctan2(p_r[..., 1:2], p_r[..., 0:1])
    return jnp.concatenate([rho, theta / jnp.pi, phi / (2 * jnp.pi) + 0.5],
                           axis=-1)


def kernel(p, x, o, params):
    P = params
    pm = _PERM
    mmat = jnp.asarray(
        (np.arange(CI)[:, None] == (pm % CI)[None, :]).astype(np.float32))

    Pq = dict(P)
    Pq['Wq'] = P['Wq'][:, pm]
    Pq['bq'] = P['bq'][pm]
    q, tab = _proj(x, Pq)
    idx = _knn(p)
    idxf = idx.reshape(-1)

    tab_i = lax.bitcast_convert_type(tab.reshape(N, TD // 2, 2), jnp.int32)
    g = _sc_gather(tab_i, idxf)
    p_r = jnp.take(p, idxf, axis=0) \
        - jnp.broadcast_to(p[:, None, :], (N, K, 3)).reshape(NK, 3)
    h_in = jnp.concatenate([p_r, _sphere(p_r)], axis=-1)

    wp2p = P['Wp2'][:, pm]
    bp2p = P['bp2'][pm]

    h, s_h, ss_h = _p1(h_in, P['Wp1'], P['bp1'])
    sc1, sh1 = _bn_coef(s_h, ss_h, NK, P['g_p1'], P['b_p1'])

    sw, ssw, sp, ssp = _p2(h, g, q, sc1, sh1, wp2p, bp2p)
    scw, shw = _bn_coef(sw, ssw, NK, P['g_fw1'][pm], P['b_fw1'][pm])
    sca, sha = _bn_coef(sp, ssp, NK, P['g_l2a'][pm], P['b_l2a'][pm])

    w1, h2, s1, ss1, s2, ss2 = _p3(h, g, q, sc1, sh1, wp2p, bp2p,
                                   scw, shw, P['Wfw1'][pm], P['bb_fw1'],
                                   sca, sha, P['Wl2'][pm], P['bl2'])
    scw2, shw2 = _bn_coef(s1, ss1, NK, P['g_fw2'], P['b_fw2'])
    scb, shb = _bn_coef(s2, ss2, NK, P['g_l2b'], P['b_l2b'])

    wpqm = jnp.concatenate([jnp.mean(P['Wpq'], axis=1).reshape(1, C),
                            jnp.full((1, 128), jnp.mean(P['bpq']), F32)],
                           axis=1)
    feat, w2, vv, sf, ssf, sw2, ssw2 = _p4(
        w1, scw2, shw2, P['Wfw2'], P['bb_fw2'], g, h, sc1, sh1,
        wp2p, bp2p, h2, scb, shb, P['Wpk'][:, pm], P['bpk'][pm], wpqm,
        g, P['Wpv'][:, pm], P['bpv'][pm], mmat)
    scf, shf = _bn_coef(sf, ssf, N, P['g_brf'][pm], P['b_brf'][pm])
    scpw, shpw = _bn_coef(sw2, ssw2, NK, P['g_pw1'][pm], P['b_pw1'][pm])

    w21, s21, ss21 = _p5(w2, scpw, shpw, P['Wpw1'][pm], P['bb_pw1'])
    scpw2, shpw2 = _bn_coef(s21, ss21, NK, P['g_pw2'], P['b_pw2'])

    post, spo, sspo = _p6(w21, scpw2, shpw2, P['Wpw2'], P['bb_pw2'], vv, mmat)
    scpo, shpo = _bn_coef(spo, sspo, N, P['g_brp'][pm], P['b_brp'][pm])

    return _p7(feat, scf, shf, post, scpo, shpo,
               P['Wfp'][0:C][pm], P['Wfp'][C:2 * C][pm], P['bfp'])
